# baseline (speedup 1.0000x reference)
import functools

import jax
import jax.numpy as jnp
from jax import lax
from jax.experimental import pallas as pl
from jax.experimental.pallas import tpu as pltpu
from jax.experimental.pallas import tpu_sc as plsc

_ANTIGEN_IDX = 2
_TEMPERATURE = 0.15

_NC, _NS, _L = 2, 16, 16
_NW = _NC * _NS
_CHUNK = 64


_T_SC = 256


def _make_sc_pool(bsz, t0, t_sc, dim):
    wpb = _NW // bsz
    tok_per_w = t_sc // wpb
    n_chunks = tok_per_w // _CHUNK
    n_groups = dim // 128

    mesh = plsc.VectorSubcoreMesh(core_axis_name="c", subcore_axis_name="s")

    def body(s_hbm, w_hbm, out_hbm, buf0, buf1, mrow, acc, sem0, sem1):
        wid = lax.axis_index("s") * _NC + lax.axis_index("c")
        b = wid // wpb
        tok0 = t0 + (wid % wpb) * tok_per_w

        pltpu.sync_copy(w_hbm.at[b, 1, pl.ds(tok0, tok_per_w)], mrow)

        zeros = jnp.zeros((_L,), jnp.float32)
        for r in range(2):
            for j in range(dim // _L):
                acc[r, pl.ds(j * _L, _L)] = zeros

        def process(buf, ci):
            mbase = ci * _CHUNK

            def group_body(g, _):
                gb = g * 128

                def sub_body(q, carry):
                    accs = list(carry)
                    fv = mrow[pl.ds(mbase + q * _L, _L)]
                    for r in range(_L):
                        f = fv[r]
                        t = q * _L + r
                        for j in range(8):
                            v = buf[t, pl.ds(gb + j * _L, _L)]
                            accs[j] = accs[j] + v
                            accs[8 + j] = accs[8 + j] + v * f
                    return tuple(accs)

                res = lax.fori_loop(0, _CHUNK // _L, sub_body, (zeros,) * 16)
                for j in range(8):
                    plsc.addupdate(acc.at[0, pl.ds(gb + j * _L, _L)], res[j])
                    plsc.addupdate(acc.at[1, pl.ds(gb + j * _L, _L)], res[8 + j])
                return 0

            lax.fori_loop(0, n_groups, group_body, 0)

        def start(buf, sem, ci):
            return pltpu.async_copy(
                s_hbm.at[b, pl.ds(tok0 + ci * _CHUNK, _CHUNK)], buf, sem)

        n_pairs = n_chunks // 2
        start(buf0, sem0, 0)

        def pair_body(p, _):
            c0 = 2 * p
            start(buf1, sem1, c0 + 1)
            pltpu.make_async_copy(
                s_hbm.at[b, pl.ds(tok0, _CHUNK)], buf0, sem0).wait()
            process(buf0, c0)
            start(buf0, sem0, jnp.minimum(c0 + 2, n_chunks - 1))
            pltpu.make_async_copy(
                s_hbm.at[b, pl.ds(tok0, _CHUNK)], buf1, sem1).wait()
            process(buf1, c0 + 1)
            return 0

        lax.fori_loop(0, n_pairs, pair_body, 0)
        pltpu.make_async_copy(
            s_hbm.at[b, pl.ds(tok0, _CHUNK)], buf0, sem0).wait()

        pltpu.sync_copy(acc, out_hbm.at[wid])

    return functools.partial(
        pl.kernel,
        out_type=jax.ShapeDtypeStruct((_NW, 2, dim), jnp.float32),
        mesh=mesh,
        scratch_types=[
            pltpu.VMEM((_CHUNK, dim), jnp.float32),
            pltpu.VMEM((_CHUNK, dim), jnp.float32),
            pltpu.VMEM((tok_per_w,), jnp.float32),
            pltpu.VMEM((2, dim), jnp.float32),
            pltpu.SemaphoreType.DMA,
            pltpu.SemaphoreType.DMA,
        ],
    )(body)


def _pool_body(w_ref, s_ref, out_ref):
    s = s_ref[0]
    n = s.shape[0]
    m = w_ref[0, 1, :].reshape(n, 1)
    tot = jnp.sum(s, axis=0)
    ag = jnp.sum(s * m, axis=0)
    out_ref[0] = jnp.stack([tot, ag], axis=0)


def _loss_body(tc_ref, sc_ref, w_ref, out_ref):
    bsz = w_ref.shape[0]
    wpb = sc_ref.shape[0] // bsz
    dim = sc_ref.shape[2]
    pooled = tc_ref[...] + jnp.sum(
        sc_ref[...].reshape(bsz, wpb, 2, dim), axis=1)
    ag_cnt = jnp.sum(w_ref[:, 1, :], axis=1)
    n_tok = w_ref.shape[2]
    ab_cnt = n_tok - ag_cnt

    tot = pooled[:, 0, :]
    ag_sum = pooled[:, 1, :]
    ab_sum = tot - ag_sum

    ab_emb = ab_sum / jnp.maximum(ab_cnt, 1.0)[:, None]
    ag_emb = ag_sum / jnp.maximum(ag_cnt, 1.0)[:, None]

    ab_n = ab_emb / jnp.maximum(
        jnp.sqrt(jnp.sum(ab_emb * ab_emb, axis=1, keepdims=True)), 1e-12)
    ag_n = ag_emb / jnp.maximum(
        jnp.sqrt(jnp.sum(ag_emb * ag_emb, axis=1, keepdims=True)), 1e-12)

    sim = jax.lax.dot_general(
        ab_n, ag_n, (((1,), (1,)), ((), ())),
        preferred_element_type=jnp.float32,
        precision=jax.lax.Precision.HIGHEST,
    ) / _TEMPERATURE

    valid = ag_cnt > 0.0
    neg_inf = jnp.asarray(-jnp.inf, dtype=sim.dtype)
    sim_m = jnp.where(valid[None, :], sim, neg_inf)
    m = jnp.max(sim_m, axis=1, keepdims=True)
    m_safe = jnp.where(jnp.isfinite(m), m, 0.0)
    lse = jnp.log(jnp.sum(jnp.exp(sim_m - m_safe), axis=1, keepdims=True)) + m

    b = sim.shape[0]
    eye = (jax.lax.broadcasted_iota(jnp.int32, (b, b), 0)
           == jax.lax.broadcasted_iota(jnp.int32, (b, b), 1))
    logp = sim - lse
    diag = jnp.sum(jnp.where(eye, logp, 0.0), axis=1)

    n_valid = jnp.sum(valid.astype(jnp.float32))
    loss = -jnp.sum(jnp.where(valid, diag, 0.0)) / n_valid
    out_ref[...] = loss[None, None]


@jax.jit
def kernel(s_i, chain_type):
    bsz, n_tok, dim = s_i.shape
    mask = (chain_type == _ANTIGEN_IDX).astype(jnp.float32)
    w = jnp.stack([jnp.ones_like(mask), mask], axis=1)

    t_tc = n_tok - _T_SC
    sc_partials = _make_sc_pool(bsz, t_tc, _T_SC, dim)(s_i, w)

    tc_pooled = pl.pallas_call(
        _pool_body,
        grid=(bsz,),
        in_specs=[
            pl.BlockSpec((1, 2, t_tc), lambda b: (b, 0, 0)),
            pl.BlockSpec((1, t_tc, dim), lambda b: (b, 0, 0)),
        ],
        out_specs=pl.BlockSpec((1, 2, dim), lambda b: (b, 0, 0)),
        out_shape=jax.ShapeDtypeStruct((bsz, 2, dim), jnp.float32),
    )(w, s_i)

    loss = pl.pallas_call(
        _loss_body,
        in_specs=[
            pl.BlockSpec(tc_pooled.shape, lambda: (0, 0, 0)),
            pl.BlockSpec(sc_partials.shape, lambda: (0, 0, 0)),
            pl.BlockSpec(w.shape, lambda: (0, 0, 0)),
        ],
        out_specs=pl.BlockSpec((1, 1), lambda: (0, 0)),
        out_shape=jax.ShapeDtypeStruct((1, 1), jnp.float32),
    )(tc_pooled, sc_partials, w)

    return loss[0, 0]


# speedup vs baseline: 1.6878x; 1.6878x over previous
import functools

import jax
import jax.numpy as jnp
from jax.experimental import pallas as pl
from jax.experimental.pallas import tpu as pltpu

_ANTIGEN_IDX = 2
_TEMPERATURE = 0.15


def _fused_body(chain_ref, s_ref, out_ref, acc_ref):
    b = pl.program_id(0)
    bsz = pl.num_programs(0)
    s = s_ref[0]
    n_tok = s.shape[0]
    chain_row = chain_ref[b, 0, :]
    m = (chain_row == _ANTIGEN_IDX).astype(jnp.float32).reshape(n_tok, 1)
    tot = jnp.sum(s, axis=0)
    ag = jnp.sum(s * m, axis=0)
    acc_ref[b] = jnp.stack([tot, ag], axis=0)

    @pl.when(b == bsz - 1)
    def _loss():
        pooled = acc_ref[...]
        mask_all = (chain_ref[:, 0, :] == _ANTIGEN_IDX).astype(jnp.float32)
        ag_cnt = jnp.sum(mask_all, axis=1)
        ab_cnt = n_tok - ag_cnt

        tot_s = pooled[:, 0, :]
        ag_s = pooled[:, 1, :]
        ab_s = tot_s - ag_s

        ab_emb = ab_s / jnp.maximum(ab_cnt, 1.0)[:, None]
        ag_emb = ag_s / jnp.maximum(ag_cnt, 1.0)[:, None]

        ab_n = ab_emb / jnp.maximum(
            jnp.sqrt(jnp.sum(ab_emb * ab_emb, axis=1, keepdims=True)), 1e-12)
        ag_n = ag_emb / jnp.maximum(
            jnp.sqrt(jnp.sum(ag_emb * ag_emb, axis=1, keepdims=True)), 1e-12)

        sim = jax.lax.dot_general(
            ab_n, ag_n, (((1,), (1,)), ((), ())),
            preferred_element_type=jnp.float32,
            precision=jax.lax.Precision.HIGHEST,
        ) / _TEMPERATURE

        valid = ag_cnt > 0.0
        neg_inf = jnp.asarray(-jnp.inf, dtype=sim.dtype)
        sim_m = jnp.where(valid[None, :], sim, neg_inf)
        mx = jnp.max(sim_m, axis=1, keepdims=True)
        mx_safe = jnp.where(jnp.isfinite(mx), mx, 0.0)
        lse = jnp.log(
            jnp.sum(jnp.exp(sim_m - mx_safe), axis=1, keepdims=True)) + mx

        eye = (jax.lax.broadcasted_iota(jnp.int32, sim.shape, 0)
               == jax.lax.broadcasted_iota(jnp.int32, sim.shape, 1))
        logp = sim - lse
        diag = jnp.sum(jnp.where(eye, logp, 0.0), axis=1)

        n_valid = jnp.sum(valid.astype(jnp.float32))
        loss = -jnp.sum(jnp.where(valid, diag, 0.0)) / n_valid
        out_ref[...] = loss[None, None]


@functools.partial(jax.jit, static_argnames=("interpret",))
def kernel(s_i, chain_type, interpret=False):
    bsz, n_tok, dim = s_i.shape
    chain3 = chain_type.reshape(bsz, 1, n_tok)

    loss = pl.pallas_call(
        _fused_body,
        grid=(bsz,),
        in_specs=[
            pl.BlockSpec((bsz, 1, n_tok), lambda b: (0, 0, 0)),
            pl.BlockSpec((1, n_tok, dim), lambda b: (b, 0, 0)),
        ],
        out_specs=pl.BlockSpec((1, 1), lambda b: (0, 0)),
        out_shape=jax.ShapeDtypeStruct((1, 1), jnp.float32),
        scratch_shapes=[pltpu.VMEM((bsz, 2, dim), jnp.float32)],
        interpret=interpret,
    )(chain3, s_i)

    return loss[0, 0]
